# Initial kernel scaffold; baseline (speedup 1.0000x reference)
import jax
import jax.numpy as jnp
from jax.experimental import pallas as pl


def kernel(hidden_states, router_logits, is_prefill, gate_up_proj, gate_up_proj_bias, down_proj, down_proj_bias):
    raise NotImplementedError("write your pallas kernel here")


# speedup vs baseline: 7.3703x; 7.3703x over previous
import functools

import jax
import jax.numpy as jnp
from jax.experimental import pallas as pl
from jax.experimental.pallas import tpu as pltpu

E = 16
K = 2
H = 1024
I = 2048
T = 2048
LIMIT = 7.0
ALPHA = 1.702

B = 128
R = T * K
P = R + E * B
MAXNB = P // B


def _moe_body(nb_ref, off_ref, xpad_ref, wg_ref, wu_ref, bg_ref, bu_ref,
              wd_ref, bd_ref, ypad_ref):
    e = pl.program_id(0)
    nb = nb_ref[e]
    off = off_ref[e]
    bg = bg_ref[0]
    bu = bu_ref[0]
    bd = bd_ref[0]

    def body(j, carry):
        @pl.when(j < nb)
        def _():
            base = pl.multiple_of(off + j * B, B)
            x = xpad_ref[pl.ds(base, B), :]
            hg = jnp.dot(x, wg_ref[0], preferred_element_type=jnp.float32) + bg
            hu = jnp.dot(x, wu_ref[0], preferred_element_type=jnp.float32) + bu
            g = jnp.minimum(hg, LIMIT)
            u = jnp.clip(hu, -LIMIT, LIMIT)
            a = (u + 1.0) * (g * jax.nn.sigmoid(g * ALPHA))
            y = jnp.dot(a.astype(jnp.bfloat16), wd_ref[0],
                        preferred_element_type=jnp.float32) + bd
            ypad_ref[pl.ds(base, B), :] = y.astype(jnp.bfloat16)
        return carry

    jax.lax.fori_loop(0, MAXNB, body, 0)


@functools.partial(jax.jit, static_argnames=())
def kernel(hidden_states, router_logits, is_prefill, gate_up_proj,
           gate_up_proj_bias, down_proj, down_proj_bias):
    del is_prefill
    probs = jax.nn.softmax(router_logits, axis=-1)
    topk_w, topk_ids = jax.lax.top_k(probs, K)
    topk_w = topk_w / jnp.sum(topk_w, axis=-1, keepdims=True)

    flat_ids = topk_ids.reshape(-1)
    order = jnp.argsort(flat_ids, stable=True)
    token_of = (order // K).astype(jnp.int32)
    expert_of = flat_ids[order]

    counts = jnp.bincount(flat_ids, length=E).astype(jnp.int32)
    nb_e = (counts + B - 1) // B
    group_start = jnp.concatenate([jnp.zeros((1,), jnp.int32),
                                   jnp.cumsum(counts)[:-1]])
    padded_off = jnp.concatenate([jnp.zeros((1,), jnp.int32),
                                  jnp.cumsum(nb_e * B)[:-1]]).astype(jnp.int32)
    r = jnp.arange(R, dtype=jnp.int32)
    ppos = padded_off[expert_of] + (r - group_start[expert_of])

    src_tok = jnp.zeros((P,), jnp.int32).at[ppos].set(token_of)
    q = jnp.zeros((R,), jnp.int32).at[order].set(ppos).reshape(T, K)

    xpad = hidden_states[src_tok].astype(jnp.bfloat16)

    wg = gate_up_proj[:, :, 0::2].astype(jnp.bfloat16)
    wu = gate_up_proj[:, :, 1::2].astype(jnp.bfloat16)
    bgv = gate_up_proj_bias[:, 0::2].reshape(E, 1, I)
    buv = gate_up_proj_bias[:, 1::2].reshape(E, 1, I)
    wd = down_proj.astype(jnp.bfloat16)
    bdv = down_proj_bias.reshape(E, 1, H)

    grid_spec = pltpu.PrefetchScalarGridSpec(
        num_scalar_prefetch=2,
        grid=(E,),
        in_specs=[
            pl.BlockSpec((P, H), lambda e, *_: (0, 0)),
            pl.BlockSpec((1, H, I), lambda e, *_: (e, 0, 0)),
            pl.BlockSpec((1, H, I), lambda e, *_: (e, 0, 0)),
            pl.BlockSpec((1, 1, I), lambda e, *_: (e, 0, 0)),
            pl.BlockSpec((1, 1, I), lambda e, *_: (e, 0, 0)),
            pl.BlockSpec((1, I, H), lambda e, *_: (e, 0, 0)),
            pl.BlockSpec((1, 1, H), lambda e, *_: (e, 0, 0)),
        ],
        out_specs=pl.BlockSpec((P, H), lambda e, *_: (0, 0)),
    )
    ypad = pl.pallas_call(
        _moe_body,
        grid_spec=grid_spec,
        out_shape=jax.ShapeDtypeStruct((P, H), jnp.bfloat16),
    )(nb_e, padded_off, xpad, wg, wu, bgv, buv, wd, bdv)

    out = (topk_w[:, 0:1] * ypad[q[:, 0]].astype(jnp.float32)
           + topk_w[:, 1:2] * ypad[q[:, 1]].astype(jnp.float32))
    return out


# speedup vs baseline: 70.7342x; 9.5972x over previous
import functools

import jax
import jax.numpy as jnp
from jax.experimental import pallas as pl
from jax.experimental.pallas import tpu as pltpu

E = 16
K = 2
H = 1024
I = 2048
T = 2048
LIMIT = 7.0
ALPHA = 1.702

B = 64
R = T * K
P = R + E * B
MAXNB = P // B
IH = I


def _moe_body(nb_ref, off_ref, xpad_ref, wgu_ref, bgu_ref, wd_ref, bd_ref,
              ypad_ref):
    e = pl.program_id(0)
    i = pl.program_id(1)
    nb = nb_ref[e]
    off = off_ref[e]
    bgu = bgu_ref[0, 0]
    bd = bd_ref[0]
    wgu = wgu_ref[0].astype(jnp.bfloat16)
    wdh = wd_ref[0, 0].astype(jnp.bfloat16)
    wd_exp = jnp.concatenate([wdh, wdh], axis=1).reshape(IH, H)
    even = (jax.lax.broadcasted_iota(jnp.int32, (1, IH), 1) % 2) == 0

    def body(j, carry):
        @pl.when(j < nb)
        def _():
            base = pl.multiple_of(off + j * B, B)
            x = xpad_ref[pl.ds(base, B), :]
            h = jnp.dot(x, wgu, preferred_element_type=jnp.float32) + bgu
            hs = pltpu.roll(h, IH - 1, 1)
            g = jnp.minimum(h, LIMIT)
            u = jnp.clip(hs, -LIMIT, LIMIT)
            val = (u + 1.0) * (g * jax.nn.sigmoid(g * ALPHA))
            a = jnp.where(even, val, 0.0).astype(jnp.bfloat16)
            y = jnp.dot(a, wd_exp, preferred_element_type=jnp.float32)

            @pl.when(i == 0)
            def _():
                ypad_ref[pl.ds(base, B), :] = y.astype(jnp.bfloat16)

            @pl.when(i == 1)
            def _():
                prev = ypad_ref[pl.ds(base, B), :].astype(jnp.float32)
                ypad_ref[pl.ds(base, B), :] = (prev + y + bd).astype(jnp.bfloat16)
        return carry

    jax.lax.fori_loop(0, MAXNB, body, 0)


@functools.partial(jax.jit, static_argnames=())
def kernel(hidden_states, router_logits, is_prefill, gate_up_proj,
           gate_up_proj_bias, down_proj, down_proj_bias):
    del is_prefill
    probs = jax.nn.softmax(router_logits, axis=-1)
    topk_w, topk_ids = jax.lax.top_k(probs, K)
    topk_w = topk_w / jnp.sum(topk_w, axis=-1, keepdims=True)

    flat_ids = topk_ids.reshape(-1)
    order = jnp.argsort(flat_ids, stable=True)
    token_of = (order // K).astype(jnp.int32)
    expert_of = flat_ids[order]

    counts = jnp.bincount(flat_ids, length=E).astype(jnp.int32)
    nb_e = (counts + B - 1) // B
    group_start = jnp.concatenate([jnp.zeros((1,), jnp.int32),
                                   jnp.cumsum(counts)[:-1]])
    padded_off = jnp.concatenate([jnp.zeros((1,), jnp.int32),
                                  jnp.cumsum(nb_e * B)[:-1]]).astype(jnp.int32)
    r = jnp.arange(R, dtype=jnp.int32)
    ppos = padded_off[expert_of] + (r - group_start[expert_of])

    src_tok = jnp.zeros((P,), jnp.int32).at[ppos].set(token_of)
    q = jnp.zeros((R,), jnp.int32).at[order].set(ppos).reshape(T, K)

    xpad = hidden_states[src_tok].astype(jnp.bfloat16)

    bgu = gate_up_proj_bias.reshape(E, 2, 1, IH)
    wdr = down_proj.reshape(E, 2, I // 2, H)
    bdv = down_proj_bias.reshape(E, 1, H)

    grid_spec = pltpu.PrefetchScalarGridSpec(
        num_scalar_prefetch=2,
        grid=(E, 2),
        in_specs=[
            pl.BlockSpec((P, H), lambda e, i, *_: (0, 0)),
            pl.BlockSpec((1, H, IH), lambda e, i, *_: (e, 0, i)),
            pl.BlockSpec((1, 1, 1, IH), lambda e, i, *_: (e, i, 0, 0)),
            pl.BlockSpec((1, 1, I // 2, H), lambda e, i, *_: (e, i, 0, 0)),
            pl.BlockSpec((1, 1, H), lambda e, i, *_: (e, 0, 0)),
        ],
        out_specs=pl.BlockSpec((P, H), lambda e, i, *_: (0, 0)),
    )
    ypad = pl.pallas_call(
        _moe_body,
        grid_spec=grid_spec,
        out_shape=jax.ShapeDtypeStruct((P, H), jnp.bfloat16),
    )(nb_e, padded_off, xpad, gate_up_proj, bgu, wdr, bdv)

    out = (topk_w[:, 0:1] * ypad[q[:, 0]].astype(jnp.float32)
           + topk_w[:, 1:2] * ypad[q[:, 1]].astype(jnp.float32))
    return out
